# baseline (speedup 1.0000x reference)
import jax
import jax.numpy as jnp
from jax.experimental import pallas as pl
from jax.experimental.pallas import tpu as pltpu

_N = 10000
_D = 128
_K = 16
_R = 200
_NP = 10240
_W = 128
_G = _NP // _W
_J = 6


def _knn_body(x_rows_ref, x_ref, sq_ref, idx_ref, dist_ref, cval_ref, cidx_ref):
    i = pl.program_id(0)
    xr = x_rows_ref[...]
    xall = x_ref[...]
    prod = jax.lax.dot_general(
        xr, xall, (((1,), (1,)), ((), ())),
        preferred_element_type=jnp.float32)
    sq_r = jnp.sum(xr * xr, axis=1, keepdims=True)
    dist = sq_r + sq_ref[...] - 2.0 * prod
    col = jax.lax.broadcasted_iota(jnp.int32, (_R, _NP), 1)
    row_g = jax.lax.broadcasted_iota(jnp.int32, (_R, _NP), 0) + i * _R
    dist_ref[...] = jnp.where(col == row_g, dist + 1e9, dist)

    lane = jax.lax.broadcasted_iota(jnp.int32, (_R, _W), 1)
    for g in range(_G):
        d = dist_ref[:, g * _W:(g + 1) * _W]
        vals, idxs = [], []
        for j in range(_J):
            m = jnp.min(d, axis=1, keepdims=True)
            a = jnp.argmin(d, axis=1).astype(jnp.int32)[:, None]
            vals.append(m)
            idxs.append(a + g * _W)
            if j < _J - 1:
                d = jnp.where(lane == a, jnp.inf, d)
        cval_ref[:, g * _J:(g + 1) * _J] = jnp.concatenate(vals, axis=1)
        cidx_ref[:, g * _J:(g + 1) * _J] = jnp.concatenate(idxs, axis=1)

    cv = cval_ref[...]
    civ = cidx_ref[...]
    clane = jax.lax.broadcasted_iota(jnp.int32, (_R, _G * _J), 1)
    deepest = (clane % _J) == (_J - 1)
    drained = jnp.zeros((_R, _G * _J), jnp.bool_)
    for k in range(_K):
        p = jnp.argmin(cv, axis=1).astype(jnp.int32)[:, None]
        sel = clane == p
        idx_ref[:, k:k + 1] = jnp.sum(jnp.where(sel, civ, 0), axis=1,
                                      keepdims=True)
        drained = drained | (sel & deepest)
        cv = jnp.where(sel, jnp.inf, cv)
    bad = jnp.any(drained)

    @pl.when(bad)
    def _repair():
        outs = []
        for _ in range(_K):
            cur = dist_ref[...]
            a = jnp.argmin(cur, axis=1).astype(jnp.int32)
            outs.append(a[:, None])
            dist_ref[...] = jnp.where(col == a[:, None], jnp.inf, cur)
        idx_ref[...] = jnp.concatenate(outs, axis=1)


def kernel(node_feat, dynamicVariable, edge_index):
    x = dynamicVariable
    x_pad = jnp.concatenate(
        [x, jnp.zeros((_NP - _N, _D), jnp.float32)], axis=0)
    sq = jnp.sum(x * x, axis=1)
    sq_pad = jnp.concatenate(
        [sq, jnp.full((_NP - _N,), 1e30, jnp.float32)])[None, :]
    idx = pl.pallas_call(
        _knn_body,
        grid=(_N // _R,),
        in_specs=[
            pl.BlockSpec((_R, _D), lambda i: (i, 0)),
            pl.BlockSpec((_NP, _D), lambda i: (0, 0)),
            pl.BlockSpec((1, _NP), lambda i: (0, 0)),
        ],
        out_specs=pl.BlockSpec((_R, _K), lambda i: (i, 0)),
        out_shape=jax.ShapeDtypeStruct((_N, _K), jnp.int32),
        scratch_shapes=[pltpu.VMEM((_R, _NP), jnp.float32),
                        pltpu.VMEM((_R, _G * _J), jnp.float32),
                        pltpu.VMEM((_R, _G * _J), jnp.int32)],
    )(x, x_pad, sq_pad)

    src = idx.reshape(-1)
    dst = jnp.repeat(jnp.arange(_N, dtype=src.dtype), _K)
    new_edges = jnp.stack([src, dst]).astype(jnp.int64)
    skip = jnp.allclose(node_feat, dynamicVariable)
    out_feat = jnp.where(skip, node_feat, dynamicVariable)
    out_edges = jnp.where(skip, edge_index, new_edges)
    return out_feat, out_edges


# speedup vs baseline: 4.7479x; 4.7479x over previous
import jax
import jax.numpy as jnp
from jax.experimental import pallas as pl
from jax.experimental.pallas import tpu as pltpu

_N = 10000
_D = 128
_K = 16
_NP = 10240
_RQ = 256
_W = 128
_G = _NP // _W
_J = 5


def _knn_body(xq_ref, x_ref, sqq_ref, idx_ref, dist_ref):
    i = pl.program_id(0)
    xq = xq_ref[...]
    xall = x_ref[...]
    prod = jax.lax.dot_general(
        xall, xq, (((1,), (1,)), ((), ())),
        preferred_element_type=jnp.float32)
    sqc = jnp.sum(xall * xall, axis=1, keepdims=True)
    sqc = jnp.where(
        jax.lax.broadcasted_iota(jnp.int32, (_NP, 1), 0) >= _N, 1e30, sqc)
    dist = sqc + sqq_ref[...] - 2.0 * prod
    srow = jax.lax.broadcasted_iota(jnp.int32, (_NP, _RQ), 0)
    qcol = jax.lax.broadcasted_iota(jnp.int32, (_NP, _RQ), 1) + i * _RQ
    dist = jnp.where(srow == qcol, dist + 1e9, dist)
    dist_ref[...] = dist

    d3 = dist.reshape(_G, _W, _RQ)
    gbase = jax.lax.broadcasted_iota(jnp.int32, (_G, _RQ), 0) * _W
    si3 = jax.lax.broadcasted_iota(jnp.int32, (_G, _W, _RQ), 1)
    cvals, cidxs = [], []
    for j in range(_J):
        m = jnp.min(d3, axis=1)
        a = jnp.argmin(d3, axis=1).astype(jnp.int32)
        cvals.append(m)
        cidxs.append(a + gbase)
        if j < _J - 1:
            d3 = jnp.where(si3 == a[:, None, :], jnp.inf, d3)

    cv = jnp.stack(cvals, axis=1).reshape(_G * _J, _RQ)
    civ = jnp.stack(cidxs, axis=1).reshape(_G * _J, _RQ)
    crow = jax.lax.broadcasted_iota(jnp.int32, (_G * _J, _RQ), 0)
    deepest = (crow % _J) == (_J - 1)
    drained = jnp.zeros((_G * _J, _RQ), jnp.bool_)
    for k in range(_K):
        p = jnp.argmin(cv, axis=0).astype(jnp.int32)
        sel = crow == p[None, :]
        idx_ref[k:k + 1, :] = jnp.sum(jnp.where(sel, civ, 0), axis=0,
                                      keepdims=True)
        drained = drained | (sel & deepest)
        cv = jnp.where(sel, jnp.inf, cv)
    bad = jnp.any(drained)

    @pl.when(bad)
    def _repair():
        for k in range(_K):
            cur = dist_ref[...]
            a = jnp.argmin(cur, axis=0).astype(jnp.int32)
            idx_ref[k:k + 1, :] = a[None, :]
            dist_ref[...] = jnp.where(srow == a[None, :], jnp.inf, cur)


def kernel(node_feat, dynamicVariable, edge_index):
    x = dynamicVariable
    x_pad = jnp.concatenate(
        [x, jnp.zeros((_NP - _N, _D), jnp.float32)], axis=0)
    sq = jnp.sum(x * x, axis=1)
    sqq_pad = jnp.concatenate(
        [sq, jnp.zeros((_NP - _N,), jnp.float32)])[None, :]
    idx_t = pl.pallas_call(
        _knn_body,
        grid=(_NP // _RQ,),
        in_specs=[
            pl.BlockSpec((_RQ, _D), lambda i: (i, 0)),
            pl.BlockSpec((_NP, _D), lambda i: (0, 0)),
            pl.BlockSpec((1, _RQ), lambda i: (0, i)),
        ],
        out_specs=pl.BlockSpec((_K, _RQ), lambda i: (0, i)),
        out_shape=jax.ShapeDtypeStruct((_K, _NP), jnp.int32),
        scratch_shapes=[pltpu.VMEM((_NP, _RQ), jnp.float32)],
    )(x_pad, x_pad, sqq_pad)

    idx = idx_t[:, :_N].T
    src = idx.reshape(-1)
    dst = jnp.repeat(jnp.arange(_N, dtype=src.dtype), _K)
    new_edges = jnp.stack([src, dst]).astype(jnp.int64)
    skip = jnp.allclose(node_feat, dynamicVariable)
    out_feat = jnp.where(skip, node_feat, dynamicVariable)
    out_edges = jnp.where(skip, edge_index, new_edges)
    return out_feat, out_edges
